# baseline (speedup 1.0000x reference)
import functools

import jax
import jax.numpy as jnp
from jax import lax
from jax.experimental import pallas as pl
from jax.experimental.pallas import tpu as pltpu
from jax.experimental.pallas import tpu_sc as plsc


CHUNK = 128


@functools.lru_cache(maxsize=None)
def _build_gather(vocab: int, dim: int, total: int):
    info = plsc.get_sparse_core_info()
    num_cores, num_subcores = info.num_cores, info.num_subcores
    num_workers = num_cores * num_subcores
    per_worker = total // num_workers
    nchunks = per_worker // CHUNK
    group = 4
    grp_rows = group * CHUNK
    npairs = nchunks // (2 * group)

    mesh = plsc.VectorSubcoreMesh(core_axis_name="c", subcore_axis_name="s")

    @functools.partial(
        pl.kernel,
        out_type=jax.ShapeDtypeStruct((total, dim), jnp.float32),
        mesh=mesh,
        compiler_params=pltpu.CompilerParams(use_tc_tiling_on_sc=False),
        scratch_types=[
            pltpu.VMEM((nchunks, CHUNK), jnp.int32),
            pltpu.VMEM((grp_rows, dim), jnp.float32),
            pltpu.VMEM((grp_rows, dim), jnp.float32),
            pltpu.SemaphoreType.DMA,
            pltpu.SemaphoreType.DMA,
        ],
    )
    def gather_kernel(table_hbm, idx_hbm, out_hbm, idx_v, buf_a, buf_b,
                      sem_a, sem_b):
        wid = lax.axis_index("s") * num_cores + lax.axis_index("c")
        row0 = wid * nchunks
        base = wid * per_worker

        pltpu.sync_copy(idx_hbm.at[pl.ds(row0, nchunks)], idx_v)

        def fire(buf, g, sem):
            for c in range(group):
                pltpu.async_copy(
                    table_hbm.at[idx_v.at[g * group + c]],
                    buf.at[pl.ds(c * CHUNK, CHUNK)],
                    sem,
                )

        def drain(buf, sem):
            pltpu.make_async_copy(
                table_hbm.at[pl.ds(0, grp_rows)], buf, sem
            ).wait()

        fire(buf_a, 0, sem_a)

        def body(gg, carry):
            g0 = 2 * gg
            fire(buf_b, g0 + 1, sem_b)
            drain(buf_a, sem_a)
            pltpu.sync_copy(
                buf_a, out_hbm.at[pl.ds(base + g0 * grp_rows, grp_rows)])

            @pl.when(gg < npairs - 1)
            def _():
                fire(buf_a, g0 + 2, sem_a)

            drain(buf_b, sem_b)
            pltpu.sync_copy(
                buf_b, out_hbm.at[pl.ds(base + (g0 + 1) * grp_rows, grp_rows)])
            return carry

        lax.fori_loop(0, npairs, body, 0)

    return gather_kernel


def kernel(word_inputs, word_seq_lengths, char_inputs, char_seq_lengths,
           char_seq_recover, word_embedding_weight):
    batch, sent_len = word_inputs.shape
    vocab, dim = word_embedding_weight.shape
    total = batch * sent_len
    idx2d = word_inputs.reshape(total // CHUNK, CHUNK)
    flat_w = jax.lax.optimization_barrier(word_embedding_weight.reshape(-1))
    table = flat_w.reshape(vocab, dim)
    gather = _build_gather(vocab, dim, total)
    out = gather(table, idx2d)
    out_flat = jax.lax.optimization_barrier(out.reshape(-1))
    return out_flat.reshape(batch, sent_len, dim)


# speedup vs baseline: 1.0022x; 1.0022x over previous
import functools

import jax
import jax.numpy as jnp
from jax import lax
from jax.experimental import pallas as pl
from jax.experimental.pallas import tpu as pltpu
from jax.experimental.pallas import tpu_sc as plsc


CHUNK = 128


@functools.lru_cache(maxsize=None)
def _build_gather(vocab: int, dim: int, total: int):
    info = plsc.get_sparse_core_info()
    num_cores, num_subcores = info.num_cores, info.num_subcores
    num_workers = num_cores * num_subcores
    per_worker = total // num_workers
    nchunks = per_worker // CHUNK
    group = 4
    grp_rows = group * CHUNK
    npairs = nchunks // (2 * group)

    mesh = plsc.VectorSubcoreMesh(core_axis_name="c", subcore_axis_name="s")

    @functools.partial(
        pl.kernel,
        out_type=jax.ShapeDtypeStruct((total, dim), jnp.float32),
        mesh=mesh,
        compiler_params=pltpu.CompilerParams(use_tc_tiling_on_sc=False),
        scratch_types=[
            pltpu.VMEM((nchunks, CHUNK), jnp.int32),
            pltpu.VMEM((grp_rows, dim), jnp.float32),
            pltpu.VMEM((grp_rows, dim), jnp.float32),
            pltpu.SemaphoreType.DMA,
            pltpu.SemaphoreType.DMA,
        ],
    )
    def gather_kernel(table_hbm, idx_hbm, out_hbm, idx_v, buf_a, buf_b,
                      sem_a, sem_b):
        wid = lax.axis_index("s") * num_cores + lax.axis_index("c")
        row0 = wid * nchunks
        base = wid * per_worker

        pltpu.sync_copy(idx_hbm.at[pl.ds(row0, nchunks)], idx_v)

        def fire(buf, g, sem):
            for c in range(group):
                pltpu.async_copy(
                    table_hbm.at[idx_v.at[g * group + c]],
                    buf.at[pl.ds(c * CHUNK, CHUNK)],
                    sem,
                )

        def drain(buf, sem):
            pltpu.make_async_copy(
                table_hbm.at[pl.ds(0, grp_rows)], buf, sem
            ).wait()

        fire(buf_a, 0, sem_a)

        def body(gg, carry):
            g0 = 2 * gg
            fire(buf_b, g0 + 1, sem_b)
            drain(buf_a, sem_a)
            pltpu.sync_copy(
                buf_a, out_hbm.at[pl.ds(base + g0 * grp_rows, grp_rows)])

            @pl.when(gg < npairs - 1)
            def _():
                fire(buf_a, g0 + 2, sem_a)

            drain(buf_b, sem_b)
            pltpu.sync_copy(
                buf_b, out_hbm.at[pl.ds(base + (g0 + 1) * grp_rows, grp_rows)])
            return carry

        lax.fori_loop(0, npairs, body, 0)

    return gather_kernel


def kernel(word_inputs, word_seq_lengths, char_inputs, char_seq_lengths,
           char_seq_recover, word_embedding_weight):
    batch, sent_len = word_inputs.shape
    vocab, dim = word_embedding_weight.shape
    total = batch * sent_len
    idx2d = word_inputs.reshape(total // CHUNK, CHUNK)
    gather = _build_gather(vocab, dim, total)
    out = gather(word_embedding_weight, idx2d)
    return out.reshape(batch, sent_len, dim)


# speedup vs baseline: 1.2189x; 1.2161x over previous
import functools

import jax
import jax.numpy as jnp
from jax import lax
from jax.experimental import pallas as pl
from jax.experimental.pallas import tpu as pltpu
from jax.experimental.pallas import tpu_sc as plsc


CHUNK = 128
PDIM = 128


@functools.lru_cache(maxsize=None)
def _build_gather(vocab: int, total: int):
    info = plsc.get_sparse_core_info()
    num_cores, num_subcores = info.num_cores, info.num_subcores
    num_workers = num_cores * num_subcores
    per_worker = total // num_workers
    nchunks = per_worker // CHUNK
    group = 2
    grp_rows = group * CHUNK
    npairs = nchunks // (2 * group)

    mesh = plsc.VectorSubcoreMesh(core_axis_name="c", subcore_axis_name="s")

    @functools.partial(
        pl.kernel,
        out_type=jax.ShapeDtypeStruct((total, PDIM), jnp.float32),
        mesh=mesh,
        compiler_params=pltpu.CompilerParams(use_tc_tiling_on_sc=True),
        scratch_types=[
            pltpu.VMEM((nchunks, CHUNK), jnp.int32),
            pltpu.VMEM((grp_rows, PDIM), jnp.float32),
            pltpu.VMEM((grp_rows, PDIM), jnp.float32),
            pltpu.SemaphoreType.DMA,
            pltpu.SemaphoreType.DMA,
        ],
    )
    def gather_kernel(table_hbm, idx_hbm, out_hbm, idx_v, buf_a, buf_b,
                      sem_a, sem_b):
        wid = lax.axis_index("s") * num_cores + lax.axis_index("c")
        row0 = wid * nchunks
        base = wid * per_worker

        pltpu.sync_copy(idx_hbm.at[pl.ds(row0, nchunks)], idx_v)

        def fire(buf, g, sem):
            for c in range(group):
                pltpu.async_copy(
                    table_hbm.at[idx_v.at[g * group + c]],
                    buf.at[pl.ds(c * CHUNK, CHUNK)],
                    sem,
                )

        def drain(buf, sem):
            pltpu.make_async_copy(
                table_hbm.at[pl.ds(0, grp_rows)], buf, sem
            ).wait()

        fire(buf_a, 0, sem_a)

        def body(gg, carry):
            g0 = 2 * gg
            fire(buf_b, g0 + 1, sem_b)
            drain(buf_a, sem_a)
            pltpu.sync_copy(
                buf_a, out_hbm.at[pl.ds(base + g0 * grp_rows, grp_rows)])

            @pl.when(gg < npairs - 1)
            def _():
                fire(buf_a, g0 + 2, sem_a)

            drain(buf_b, sem_b)
            pltpu.sync_copy(
                buf_b, out_hbm.at[pl.ds(base + (g0 + 1) * grp_rows, grp_rows)])
            return carry

        lax.fori_loop(0, npairs, body, 0)

    return gather_kernel


def kernel(word_inputs, word_seq_lengths, char_inputs, char_seq_lengths,
           char_seq_recover, word_embedding_weight):
    batch, sent_len = word_inputs.shape
    vocab, dim = word_embedding_weight.shape
    total = batch * sent_len
    idx2d = word_inputs.reshape(total // CHUNK, CHUNK)
    table_padded = jnp.pad(word_embedding_weight, ((0, 0), (0, PDIM - dim)))
    gather = _build_gather(vocab, total)
    out = gather(table_padded, idx2d)
    return out[:, :dim].reshape(batch, sent_len, dim)
